# baseline (speedup 1.0000x reference)
import jax
import jax.numpy as jnp
from jax.experimental import pallas as pl
from jax.experimental.pallas import tpu as pltpu


def _routing_body(nt, ntok, x_ref, tp_ref, wgi_ref, wgt_ref, bg_ref,
                  wd_ref, laux_ref, acc_ref):
    t = pl.program_id(0)
    logits = (jnp.dot(x_ref[...], wgi_ref[...], preferred_element_type=jnp.float32)
              + jnp.dot(tp_ref[...], wgt_ref[...], preferred_element_type=jnp.float32)
              + bg_ref[...])
    ne = logits.shape[1]
    lane = jax.lax.broadcasted_iota(jnp.int32, logits.shape, 1)
    m1 = jnp.max(logits, axis=1, keepdims=True)
    i1 = jnp.min(jnp.where(logits == m1, lane, ne), axis=1, keepdims=True)
    masked = jnp.where(lane == i1, -jnp.inf, logits)
    m2 = jnp.max(masked, axis=1, keepdims=True)
    i2 = jnp.min(jnp.where(masked == m2, lane, ne), axis=1, keepdims=True)
    s = jnp.exp(m2 - m1)
    w1 = 1.0 / (1.0 + s)
    w2 = s / (1.0 + s)
    sel1 = lane == i1
    sel2 = lane == i2
    wd_ref[...] = jnp.where(sel1, w1, 0.0) + jnp.where(sel2, w2, 0.0)
    p = jnp.exp(logits - m1)
    p = p / jnp.sum(p, axis=1, keepdims=True)
    cnt = sel1.astype(jnp.float32) + sel2.astype(jnp.float32)

    @pl.when(t == 0)
    def _():
        acc_ref[...] = jnp.zeros_like(acc_ref)

    acc_ref[0:1, :] += jnp.sum(p, axis=0, keepdims=True)
    acc_ref[1:2, :] += jnp.sum(cnt, axis=0, keepdims=True)

    @pl.when(t == nt - 1)
    def _():
        laux_ref[...] = (jnp.sum(acc_ref[0:1, :] * acc_ref[1:2, :])
                         / (ntok * ntok)).reshape(1, 1)


def _ffn_body(x_ref, w1_ref, b1_ref, w2_ref, b2_ref, wd_ref, out_ref):
    e = pl.program_id(1)
    f = pl.program_id(2)

    @pl.when((e == 0) & (f == 0))
    def _():
        out_ref[...] = jnp.zeros_like(out_ref)

    h = jnp.dot(x_ref[...], w1_ref[0], preferred_element_type=jnp.float32) + b1_ref[0]
    h = jnp.maximum(h, 0.0).astype(w2_ref.dtype)
    contrib = jnp.dot(h, w2_ref[0], preferred_element_type=jnp.float32)
    wd = wd_ref[...]
    lane = jax.lax.broadcasted_iota(jnp.int32, wd.shape, 1)
    wcol = jnp.sum(jnp.where(lane == e, wd, 0.0), axis=1, keepdims=True)

    @pl.when(f == 0)
    def _():
        out_ref[...] += wcol * b2_ref[0]

    out_ref[...] += wcol * contrib


def kernel(inputs, task_param, alpha, Wg_in, bg_in, Wg_task, bg_task, W1, b1, W2, b2):
    bsz, seq, dm = inputs.shape
    ne = Wg_in.shape[1]
    fd = W1.shape[2]
    n = bsz * seq

    xf = inputs.reshape(n, dm)
    tf = task_param.reshape(n, dm)
    a = alpha.astype(jnp.float32)
    wgi = (1.0 - a) * Wg_in
    wgt = a * Wg_task
    bg = ((1.0 - a) * bg_in + a * bg_task).reshape(1, ne)

    mr = min(1024, n)
    nt = n // mr
    wd, laux = pl.pallas_call(
        lambda *refs: _routing_body(nt, n, *refs),
        grid=(nt,),
        in_specs=[
            pl.BlockSpec((mr, dm), lambda t: (t, 0)),
            pl.BlockSpec((mr, dm), lambda t: (t, 0)),
            pl.BlockSpec((dm, ne), lambda t: (0, 0)),
            pl.BlockSpec((dm, ne), lambda t: (0, 0)),
            pl.BlockSpec((1, ne), lambda t: (0, 0)),
        ],
        out_specs=[
            pl.BlockSpec((mr, ne), lambda t: (t, 0)),
            pl.BlockSpec((1, 1), lambda t: (0, 0)),
        ],
        out_shape=[
            jax.ShapeDtypeStruct((n, ne), jnp.float32),
            jax.ShapeDtypeStruct((1, 1), jnp.float32),
        ],
        scratch_shapes=[pltpu.VMEM((2, ne), jnp.float32)],
    )(xf, tf, wgi, wgt, bg)

    mt = min(1024, n)
    fb = min(512, fd)
    res = pl.pallas_call(
        _ffn_body,
        grid=(n // mt, ne, fd // fb),
        in_specs=[
            pl.BlockSpec((mt, dm), lambda t, e, f: (t, 0)),
            pl.BlockSpec((1, dm, fb), lambda t, e, f: (e, 0, f)),
            pl.BlockSpec((1, 1, fb), lambda t, e, f: (e, 0, f)),
            pl.BlockSpec((1, fb, dm), lambda t, e, f: (e, f, 0)),
            pl.BlockSpec((1, 1, dm), lambda t, e, f: (e, 0, 0)),
            pl.BlockSpec((mt, ne), lambda t, e, f: (t, 0)),
        ],
        out_specs=pl.BlockSpec((mt, dm), lambda t, e, f: (t, 0)),
        out_shape=jax.ShapeDtypeStruct((n, dm), jnp.float32),
    )(xf.astype(jnp.bfloat16), W1.astype(jnp.bfloat16),
      b1.reshape(ne, 1, fd), W2.astype(jnp.bfloat16),
      b2.reshape(ne, 1, dm), wd)

    return res.reshape(bsz, seq, dm), laux[0, 0]


# speedup vs baseline: 1.2326x; 1.2326x over previous
import jax
import jax.numpy as jnp
from jax.experimental import pallas as pl
from jax.experimental.pallas import tpu as pltpu


def _routing_body(nt, ntok, x_ref, tp_ref, wgi_ref, wgt_ref, bg_ref,
                  wd_ref, laux_ref, acc_ref):
    t = pl.program_id(0)
    logits = (jnp.dot(x_ref[...], wgi_ref[...], preferred_element_type=jnp.float32)
              + jnp.dot(tp_ref[...], wgt_ref[...], preferred_element_type=jnp.float32)
              + bg_ref[...])
    ne = logits.shape[1]
    lane = jax.lax.broadcasted_iota(jnp.int32, logits.shape, 1)
    m1 = jnp.max(logits, axis=1, keepdims=True)
    i1 = jnp.min(jnp.where(logits == m1, lane, ne), axis=1, keepdims=True)
    masked = jnp.where(lane == i1, -jnp.inf, logits)
    m2 = jnp.max(masked, axis=1, keepdims=True)
    i2 = jnp.min(jnp.where(masked == m2, lane, ne), axis=1, keepdims=True)
    s = jnp.exp(m2 - m1)
    w1 = 1.0 / (1.0 + s)
    w2 = s / (1.0 + s)
    sel1 = lane == i1
    sel2 = lane == i2
    wd_ref[...] = jnp.where(sel1, w1, 0.0) + jnp.where(sel2, w2, 0.0)
    p = jnp.exp(logits - m1)
    p = p / jnp.sum(p, axis=1, keepdims=True)
    cnt = sel1.astype(jnp.float32) + sel2.astype(jnp.float32)

    @pl.when(t == 0)
    def _():
        acc_ref[...] = jnp.zeros_like(acc_ref)

    acc_ref[0:1, :] += jnp.sum(p, axis=0, keepdims=True)
    acc_ref[1:2, :] += jnp.sum(cnt, axis=0, keepdims=True)

    @pl.when(t == nt - 1)
    def _():
        laux_ref[...] = (jnp.sum(acc_ref[0:1, :] * acc_ref[1:2, :])
                         / (ntok * ntok)).reshape(1, 1)


def _ffn_body(x_ref, w1_ref, b1_ref, w2_ref, b2_ref, wd_ref, out_ref):
    e = pl.program_id(1)

    @pl.when(e == 0)
    def _():
        out_ref[...] = jnp.zeros_like(out_ref)

    wd = wd_ref[...]
    lane = jax.lax.broadcasted_iota(jnp.int32, wd.shape, 1)
    wcol = jnp.sum(jnp.where(lane == e, wd, 0.0), axis=1, keepdims=True)

    h = jnp.dot(x_ref[...], w1_ref[0], preferred_element_type=jnp.float32) + b1_ref[0]
    h = (wcol * jnp.maximum(h, 0.0)).astype(w2_ref.dtype)
    contrib = jnp.dot(h, w2_ref[0], preferred_element_type=jnp.float32)
    out_ref[...] += contrib + wcol * b2_ref[0]


def kernel(inputs, task_param, alpha, Wg_in, bg_in, Wg_task, bg_task, W1, b1, W2, b2):
    bsz, seq, dm = inputs.shape
    ne = Wg_in.shape[1]
    fd = W1.shape[2]
    n = bsz * seq

    xf = inputs.reshape(n, dm)
    tf = task_param.reshape(n, dm)
    a = alpha.astype(jnp.float32)
    wgi = (1.0 - a) * Wg_in
    wgt = a * Wg_task
    bg = ((1.0 - a) * bg_in + a * bg_task).reshape(1, ne)

    mr = min(1024, n)
    nt = n // mr
    wd, laux = pl.pallas_call(
        lambda *refs: _routing_body(nt, n, *refs),
        grid=(nt,),
        in_specs=[
            pl.BlockSpec((mr, dm), lambda t: (t, 0)),
            pl.BlockSpec((mr, dm), lambda t: (t, 0)),
            pl.BlockSpec((dm, ne), lambda t: (0, 0)),
            pl.BlockSpec((dm, ne), lambda t: (0, 0)),
            pl.BlockSpec((1, ne), lambda t: (0, 0)),
        ],
        out_specs=[
            pl.BlockSpec((mr, ne), lambda t: (t, 0)),
            pl.BlockSpec((1, 1), lambda t: (0, 0)),
        ],
        out_shape=[
            jax.ShapeDtypeStruct((n, ne), jnp.float32),
            jax.ShapeDtypeStruct((1, 1), jnp.float32),
        ],
        scratch_shapes=[pltpu.VMEM((2, ne), jnp.float32)],
    )(xf, tf, wgi, wgt, bg)

    mt = min(1024, n)
    res = pl.pallas_call(
        _ffn_body,
        grid=(n // mt, ne),
        in_specs=[
            pl.BlockSpec((mt, dm), lambda t, e: (t, 0)),
            pl.BlockSpec((1, dm, fd), lambda t, e: (e, 0, 0)),
            pl.BlockSpec((1, 1, fd), lambda t, e: (e, 0, 0)),
            pl.BlockSpec((1, fd, dm), lambda t, e: (e, 0, 0)),
            pl.BlockSpec((1, 1, dm), lambda t, e: (e, 0, 0)),
            pl.BlockSpec((mt, ne), lambda t, e: (t, 0)),
        ],
        out_specs=pl.BlockSpec((mt, dm), lambda t, e: (t, 0)),
        out_shape=jax.ShapeDtypeStruct((n, dm), jnp.float32),
    )(xf.astype(jnp.bfloat16), W1.astype(jnp.bfloat16),
      b1.reshape(ne, 1, fd), W2.astype(jnp.bfloat16),
      b2.reshape(ne, 1, dm), wd)

    return res.reshape(bsz, seq, dm), laux[0, 0]
